# Initial kernel scaffold; baseline (speedup 1.0000x reference)
import jax
import jax.numpy as jnp
from jax.experimental import pallas as pl


def kernel(h, edge_index0, edge_index1, W00, b00, W01, b01, W10, b10, W11, b11, Wp, bp, Wq):
    raise NotImplementedError("write your pallas kernel here")


# speedup vs baseline: 6.1212x; 6.1212x over previous
import functools

import jax
import jax.numpy as jnp
from jax import lax
from jax.experimental import pallas as pl
from jax.experimental.pallas import tpu as pltpu
from jax.experimental.pallas import tpu_sc as plsc

N = 10000
E = 320000
IN = 128
HID = 1000
HIDP = 1024
OUT = 256
ATT_H = 128

NP = 10240

NC = 2
NS = 16
NW = NC * NS
EPW = E // NW
B = 80
NB = EPW // B
RPS = NP // NS

DL = 128

RB = 400
GN = N // RB

@functools.cache
def _mesh():
    return plsc.VectorSubcoreMesh(
        core_axis_name="c", subcore_axis_name="s",
        num_cores=NC, num_subcores=NS)


def _deg_body(idx_hbm, ones_hbm, z128_hbm, degp_hbm, acc, idx_v, ones_v):
    c = lax.axis_index("c")
    s = lax.axis_index("s")
    w = c * NS + s
    r0 = s * RPS
    pltpu.sync_copy(ones_hbm, ones_v)
    for j in range(4):
        pltpu.sync_copy(z128_hbm.at[pl.ds(r0, RPS), :],
                        acc.at[pl.ds(r0, RPS), :])
        plsc.subcore_barrier()

        def body(b, carry, j=j):
            pltpu.sync_copy(idx_hbm.at[j, w, b], idx_v)
            pltpu.sync_copy(ones_v, acc.at[idx_v], add=True)
            return carry

        lax.fori_loop(0, NB, body, 0)
        plsc.subcore_barrier()
        pltpu.sync_copy(acc.at[pl.ds(r0, RPS), :],
                        degp_hbm.at[c, j, pl.ds(r0, RPS), :])
        plsc.subcore_barrier()


@functools.cache
def _deg_kernel():
    return pl.kernel(
        _deg_body,
        out_type=jax.ShapeDtypeStruct((NC, 4, NP, DL), jnp.float32),
        mesh=_mesh(),
        scratch_types=[
            pltpu.VMEM_SHARED((NP, DL), jnp.float32),
            pltpu.VMEM((B,), jnp.int32),
            pltpu.VMEM((B, DL), jnp.float32),
        ],
    )


def _agg_body(table_hbm, src_hbm, dst_hbm, zeros_hbm, part_hbm,
              acc, idx_s, idx_d, rows, sem):
    c = lax.axis_index("c")
    s = lax.axis_index("s")
    w = c * NS + s
    r0 = s * RPS
    pltpu.sync_copy(zeros_hbm.at[pl.ds(r0, RPS), :],
                    acc.at[pl.ds(r0, RPS), :])
    plsc.subcore_barrier()

    def body(b, carry):
        pltpu.sync_copy(src_hbm.at[w, b], idx_s)
        pltpu.sync_copy(dst_hbm.at[w, b], idx_d)
        pltpu.async_copy(table_hbm.at[idx_s], rows, sem).wait()
        pltpu.sync_copy(rows, acc.at[idx_d], add=True)
        return carry

    lax.fori_loop(0, NB, body, 0)
    plsc.subcore_barrier()
    pltpu.sync_copy(acc.at[pl.ds(r0, RPS), :],
                    part_hbm.at[c, pl.ds(r0, RPS), :])


@functools.cache
def _agg_kernel():
    return pl.kernel(
        _agg_body,
        out_type=jax.ShapeDtypeStruct((NC, NP, IN), jnp.float32),
        mesh=_mesh(),
        scratch_types=[
            pltpu.VMEM_SHARED((NP, IN), jnp.float32),
            pltpu.VMEM((B,), jnp.int32),
            pltpu.VMEM((B,), jnp.int32),
            pltpu.VMEM((B, IN), jnp.float32),
            pltpu.SemaphoreType.DMA,
        ],
    )


def _nrm(x):
    return jnp.where(x > 0, lax.rsqrt(x), 0.0)


def _prep_body(degp_ref, h_ref, xs0_ref, xs1_ref, norms_ref):
    d = degp_ref[0] + degp_ref[1]
    ns0 = _nrm(d[0])
    nd0 = _nrm(d[1])
    ns1 = _nrm(d[2])
    nd1 = _nrm(d[3])
    h = h_ref[...]
    xs0_ref[...] = h * ns0[:, :1]
    xs1_ref[...] = h * ns1[:, :1]
    norms_ref[...] = jnp.stack([ns0, nd0, ns1, nd1], axis=0)


def _branch_body(part_ref, norms_ref, w0_ref, b0_ref, w1_ref,
                 y0_ref, y1_ref, *, m):
    nd = norms_ref[2 * m + 1][:, :1]
    ns = norms_ref[2 * m][:, :1]
    agg = (part_ref[0] + part_ref[1]) * nd
    z = jnp.dot(agg, w0_ref[...], preferred_element_type=jnp.float32)
    z = jnp.maximum(z + b0_ref[...], 0.0)
    y = jnp.dot(z * ns, w1_ref[...], preferred_element_type=jnp.float32)
    y0_ref[...] = y[:, :IN]
    y1_ref[...] = y[:, IN:]


def _att_body(q00_ref, q01_ref, q10_ref, q11_ref, norms_ref,
              b01_ref, b11_ref, wp_ref, bp_ref, wqr_ref,
              e0_ref, e1_ref, sc_ref):
    i = pl.program_id(0)
    nd0 = norms_ref[1][:, :1]
    nd1 = norms_ref[3][:, :1]
    e0 = jnp.concatenate(
        [q00_ref[0] + q00_ref[1], q01_ref[0] + q01_ref[1]], axis=1)
    e0 = e0 * nd0 + b01_ref[...]
    e1 = jnp.concatenate(
        [q10_ref[0] + q10_ref[1], q11_ref[0] + q11_ref[1]], axis=1)
    e1 = e1 * nd1 + b11_ref[...]
    e0_ref[...] = e0
    e1_ref[...] = e1
    t0 = jnp.tanh(jnp.dot(e0, wp_ref[...],
                          preferred_element_type=jnp.float32) + bp_ref[...])
    t1 = jnp.tanh(jnp.dot(e1, wp_ref[...],
                          preferred_element_type=jnp.float32) + bp_ref[...])
    s0 = jnp.sum(t0 * wqr_ref[...])
    s1 = jnp.sum(t1 * wqr_ref[...])
    srow = jnp.concatenate(
        [jnp.full((1, 128), s0, jnp.float32),
         jnp.full((1, 128), s1, jnp.float32)], axis=0)

    @pl.when(i == 0)
    def _():
        sc_ref[...] = srow

    @pl.when(i > 0)
    def _():
        sc_ref[...] = sc_ref[...] + srow


def _mix_body(e0_ref, e1_ref, sc_ref, out_ref):
    w0 = jnp.max(sc_ref[0]) * (1.0 / N)
    w1 = jnp.max(sc_ref[1]) * (1.0 / N)
    mx = jnp.maximum(w0, w1)
    a = jnp.exp(w0 - mx)
    b = jnp.exp(w1 - mx)
    beta0 = a / (a + b)
    out_ref[...] = beta0 * e0_ref[...] + (1.0 - beta0) * e1_ref[...]


_prep_call = pl.pallas_call(
    _prep_body,
    grid=(GN,),
    in_specs=[
        pl.BlockSpec((NC, 4, RB, DL), lambda i: (0, 0, i, 0)),
        pl.BlockSpec((RB, IN), lambda i: (i, 0)),
    ],
    out_specs=[
        pl.BlockSpec((RB, IN), lambda i: (i, 0)),
        pl.BlockSpec((RB, IN), lambda i: (i, 0)),
        pl.BlockSpec((4, RB, DL), lambda i: (0, i, 0)),
    ],
    out_shape=[
        jax.ShapeDtypeStruct((N, IN), jnp.float32),
        jax.ShapeDtypeStruct((N, IN), jnp.float32),
        jax.ShapeDtypeStruct((4, N, DL), jnp.float32),
    ],
)


def _branch_call(m):
    return pl.pallas_call(
        functools.partial(_branch_body, m=m),
        grid=(GN,),
        in_specs=[
            pl.BlockSpec((NC, RB, IN), lambda i: (0, i, 0)),
            pl.BlockSpec((4, RB, DL), lambda i: (0, i, 0)),
            pl.BlockSpec((IN, HIDP), lambda i: (0, 0)),
            pl.BlockSpec((1, HIDP), lambda i: (0, 0)),
            pl.BlockSpec((HIDP, OUT), lambda i: (0, 0)),
        ],
        out_specs=[
            pl.BlockSpec((RB, IN), lambda i: (i, 0)),
            pl.BlockSpec((RB, IN), lambda i: (i, 0)),
        ],
        out_shape=[
            jax.ShapeDtypeStruct((N, IN), jnp.float32),
            jax.ShapeDtypeStruct((N, IN), jnp.float32),
        ],
    )


_att_call = pl.pallas_call(
    _att_body,
    grid=(GN,),
    in_specs=[
        pl.BlockSpec((NC, RB, IN), lambda i: (0, i, 0)),
        pl.BlockSpec((NC, RB, IN), lambda i: (0, i, 0)),
        pl.BlockSpec((NC, RB, IN), lambda i: (0, i, 0)),
        pl.BlockSpec((NC, RB, IN), lambda i: (0, i, 0)),
        pl.BlockSpec((4, RB, DL), lambda i: (0, i, 0)),
        pl.BlockSpec((1, OUT), lambda i: (0, 0)),
        pl.BlockSpec((1, OUT), lambda i: (0, 0)),
        pl.BlockSpec((OUT, ATT_H), lambda i: (0, 0)),
        pl.BlockSpec((1, ATT_H), lambda i: (0, 0)),
        pl.BlockSpec((1, ATT_H), lambda i: (0, 0)),
    ],
    out_specs=[
        pl.BlockSpec((RB, OUT), lambda i: (i, 0)),
        pl.BlockSpec((RB, OUT), lambda i: (i, 0)),
        pl.BlockSpec((2, 128), lambda i: (0, 0)),
    ],
    out_shape=[
        jax.ShapeDtypeStruct((N, OUT), jnp.float32),
        jax.ShapeDtypeStruct((N, OUT), jnp.float32),
        jax.ShapeDtypeStruct((2, 128), jnp.float32),
    ],
)

_mix_call = pl.pallas_call(
    _mix_body,
    grid=(GN,),
    in_specs=[
        pl.BlockSpec((RB, OUT), lambda i: (i, 0)),
        pl.BlockSpec((RB, OUT), lambda i: (i, 0)),
        pl.BlockSpec((2, 128), lambda i: (0, 0)),
    ],
    out_specs=pl.BlockSpec((RB, OUT), lambda i: (i, 0)),
    out_shape=jax.ShapeDtypeStruct((N, OUT), jnp.float32),
)


def kernel(h, edge_index0, edge_index1,
           W00, b00, W01, b01, W10, b10, W11, b11, Wp, bp, Wq):
    f32 = jnp.float32
    src0 = edge_index0[0].reshape(NW, NB, B)
    dst0 = edge_index0[1].reshape(NW, NB, B)
    src1 = edge_index1[0].reshape(NW, NB, B)
    dst1 = edge_index1[1].reshape(NW, NB, B)
    idx4 = jnp.stack([src0, dst0, src1, dst1])
    ones128 = jnp.ones((B, DL), f32)
    z128 = jnp.zeros((NP, IN), f32)

    degp = _deg_kernel()(idx4, ones128, z128)
    xs0, xs1, norms = _prep_call(degp, h)

    p0 = _agg_kernel()(xs0, src0, dst0, z128)
    p1 = _agg_kernel()(xs1, src1, dst1, z128)

    W0p0 = jnp.pad(W00, ((0, 0), (0, HIDP - HID)))
    b0p0 = jnp.pad(b00, (0, HIDP - HID)).reshape(1, HIDP)
    W1p0 = jnp.pad(W01, ((0, HIDP - HID), (0, 0)))
    W0p1 = jnp.pad(W10, ((0, 0), (0, HIDP - HID)))
    b0p1 = jnp.pad(b10, (0, HIDP - HID)).reshape(1, HIDP)
    W1p1 = jnp.pad(W11, ((0, HIDP - HID), (0, 0)))

    y00, y01 = _branch_call(0)(p0, norms, W0p0, b0p0, W1p0)
    y10, y11 = _branch_call(1)(p1, norms, W0p1, b0p1, W1p1)

    q00 = _agg_kernel()(y00, src0, dst0, z128)
    q01 = _agg_kernel()(y01, src0, dst0, z128)
    q10 = _agg_kernel()(y10, src1, dst1, z128)
    q11 = _agg_kernel()(y11, src1, dst1, z128)

    e0, e1, scores = _att_call(
        q00, q01, q10, q11, norms,
        b01.reshape(1, OUT), b11.reshape(1, OUT),
        Wp, bp.reshape(1, ATT_H), Wq.reshape(1, ATT_H))
    return _mix_call(e0, e1, scores)
